# Initial kernel scaffold; baseline (speedup 1.0000x reference)
import jax
import jax.numpy as jnp
from jax.experimental import pallas as pl


def kernel(x, codebook):
    raise NotImplementedError("write your pallas kernel here")


# speedup vs baseline: 2.1125x; 2.1125x over previous
import jax
import jax.numpy as jnp
from jax import lax
from jax.experimental import pallas as pl

NUM_CODES = 1024
DIM = 64
PIX = 1024


def _vq_kernel(x_ref, cb_ref, idx_ref, q_ref):
    xb = x_ref[0]
    cb = cb_ref[...]

    cnorm = jnp.sum(cb * cb, axis=1, keepdims=True)
    xnorm = jnp.sum(xb * xb, axis=0, keepdims=True)
    cross = lax.dot_general(cb, xb, (((1,), (0,)), ((), ())),
                            preferred_element_type=jnp.float32)
    dist = (xnorm + cnorm) - 2.0 * cross

    minval = jnp.min(dist, axis=0, keepdims=True)
    rowid = lax.broadcasted_iota(jnp.int32, dist.shape, 0)
    idx = jnp.min(jnp.where(dist == minval, rowid, NUM_CODES),
                  axis=0, keepdims=True)

    onehot = (rowid == idx).astype(jnp.float32)
    q = lax.dot_general(cb, onehot, (((0,), (0,)), ((), ())),
                        preferred_element_type=jnp.float32)

    idx_ref[0] = idx
    q_ref[0] = q


def kernel(x, codebook):
    B, C, H, W = x.shape
    xflat = x.reshape(B, C, H * W)
    idx, q = pl.pallas_call(
        _vq_kernel,
        grid=(B,),
        in_specs=[
            pl.BlockSpec((1, C, H * W), lambda b: (b, 0, 0)),
            pl.BlockSpec((NUM_CODES, DIM), lambda b: (0, 0)),
        ],
        out_specs=[
            pl.BlockSpec((1, 1, H * W), lambda b: (b, 0, 0)),
            pl.BlockSpec((1, C, H * W), lambda b: (b, 0, 0)),
        ],
        out_shape=[
            jax.ShapeDtypeStruct((B, 1, H * W), jnp.int32),
            jax.ShapeDtypeStruct((B, C, H * W), jnp.float32),
        ],
    )(xflat, codebook)
    indices = idx.reshape(B * H * W, 1)
    quantized = q.reshape(B, C, H, W)
    return (indices, quantized)


# speedup vs baseline: 2.4306x; 1.1506x over previous
import jax
import jax.numpy as jnp
from jax import lax
from jax.experimental import pallas as pl

NUM_CODES = 1024
DIM = 64
PIX = 1024


def _vq_kernel(x_ref, cb_ref, idx_ref, q_ref):
    xb = x_ref[0]
    cb = cb_ref[...]

    cnorm = jnp.sum(cb * cb, axis=1, keepdims=True)
    xnorm = jnp.sum(xb * xb, axis=0, keepdims=True)
    cross = lax.dot_general(cb, xb, (((1,), (0,)), ((), ())),
                            preferred_element_type=jnp.float32)
    dist = (xnorm + cnorm) - 2.0 * cross

    idx = jnp.argmin(dist, axis=0)[None, :]
    rowid = lax.broadcasted_iota(jnp.int32, dist.shape, 0)

    onehot = (rowid == idx).astype(jnp.float32)
    q = lax.dot_general(cb, onehot, (((0,), (0,)), ((), ())),
                        preferred_element_type=jnp.float32)

    idx_ref[0] = idx
    q_ref[0] = q


def kernel(x, codebook):
    B, C, H, W = x.shape
    xflat = x.reshape(B, C, H * W)
    idx, q = pl.pallas_call(
        _vq_kernel,
        grid=(B,),
        in_specs=[
            pl.BlockSpec((1, C, H * W), lambda b: (b, 0, 0)),
            pl.BlockSpec((NUM_CODES, DIM), lambda b: (0, 0)),
        ],
        out_specs=[
            pl.BlockSpec((1, 1, H * W), lambda b: (b, 0, 0)),
            pl.BlockSpec((1, C, H * W), lambda b: (b, 0, 0)),
        ],
        out_shape=[
            jax.ShapeDtypeStruct((B, 1, H * W), jnp.int32),
            jax.ShapeDtypeStruct((B, C, H * W), jnp.float32),
        ],
    )(xflat, codebook)
    indices = idx.reshape(B * H * W, 1)
    quantized = q.reshape(B, C, H, W)
    return (indices, quantized)


# speedup vs baseline: 2.5618x; 1.0540x over previous
import jax
import jax.numpy as jnp
from jax import lax
from jax.experimental import pallas as pl

NUM_CODES = 1024
DIM = 64
PIX = 1024


BATCH_BLOCK = 2


def _vq_kernel(x_ref, cb_ref, idx_ref, q_ref):
    xb = jnp.concatenate([x_ref[i] for i in range(BATCH_BLOCK)], axis=1)
    cb = cb_ref[...]

    cnorm = jnp.sum(cb * cb, axis=1, keepdims=True)
    xnorm = jnp.sum(xb * xb, axis=0, keepdims=True)
    cross = lax.dot_general(cb, xb, (((1,), (0,)), ((), ())),
                            preferred_element_type=jnp.float32)
    dist = (xnorm + cnorm) - 2.0 * cross

    idx = jnp.argmin(dist, axis=0)[None, :]
    rowid = lax.broadcasted_iota(jnp.int32, dist.shape, 0)

    onehot = (rowid == idx).astype(jnp.float32)
    q = lax.dot_general(cb, onehot, (((0,), (0,)), ((), ())),
                        preferred_element_type=jnp.float32)

    for i in range(BATCH_BLOCK):
        idx_ref[i] = idx[:, i * PIX:(i + 1) * PIX]
        q_ref[i] = q[:, i * PIX:(i + 1) * PIX]


def kernel(x, codebook):
    B, C, H, W = x.shape
    xflat = x.reshape(B, C, H * W)
    nb = B // BATCH_BLOCK
    idx, q = pl.pallas_call(
        _vq_kernel,
        grid=(nb,),
        in_specs=[
            pl.BlockSpec((BATCH_BLOCK, C, H * W), lambda b: (b, 0, 0)),
            pl.BlockSpec((NUM_CODES, DIM), lambda b: (0, 0)),
        ],
        out_specs=[
            pl.BlockSpec((BATCH_BLOCK, 1, H * W), lambda b: (b, 0, 0)),
            pl.BlockSpec((BATCH_BLOCK, C, H * W), lambda b: (b, 0, 0)),
        ],
        out_shape=[
            jax.ShapeDtypeStruct((B, 1, H * W), jnp.int32),
            jax.ShapeDtypeStruct((B, C, H * W), jnp.float32),
        ],
    )(xflat, codebook)
    indices = idx.reshape(B * H * W, 1)
    quantized = q.reshape(B, C, H, W)
    return (indices, quantized)
